# Initial kernel scaffold; baseline (speedup 1.0000x reference)
import jax
import jax.numpy as jnp
from jax.experimental import pallas as pl


def kernel(h, edge_index):
    raise NotImplementedError("write your pallas kernel here")


# speedup vs baseline: 2.5780x; 2.5780x over previous
import functools

import jax
import jax.numpy as jnp
from jax import lax
from jax.experimental import pallas as pl
from jax.experimental.pallas import tpu as pltpu
from jax.experimental.pallas import tpu_sc as plsc

N_NODES = 10000
N_EDGES = 320000
D_FEAT = 128

NUM_CORES = 2
NUM_SUBCORES = 16
LANES = 16
NW = NUM_CORES * NUM_SUBCORES
E_PER_W = N_EDGES // NW
CHUNK = 80
N_CHUNKS = E_PER_W // CHUNK
GROUPS = CHUNK // LANES

_mesh = plsc.VectorSubcoreMesh(core_axis_name="c", subcore_axis_name="s")


@functools.partial(
    pl.kernel,
    mesh=_mesh,
    compiler_params=pltpu.CompilerParams(needs_layout_passes=False),
    out_type=jax.ShapeDtypeStruct((N_EDGES,), jnp.float32),
    scratch_types=[
        pltpu.VMEM((CHUNK,), jnp.int32),
        pltpu.VMEM((CHUNK,), jnp.int32),
        pltpu.VMEM((CHUNK, D_FEAT), jnp.float32),
        pltpu.VMEM((CHUNK, D_FEAT), jnp.float32),
        pltpu.VMEM((CHUNK,), jnp.float32),
        pltpu.SemaphoreType.DMA,
        pltpu.SemaphoreType.DMA,
    ],
)
def _edge_dot_kernel(h_hbm, src_hbm, dst_hbm, out_hbm,
                     idx_u, idx_v, rows_u, rows_v, out_v, sem_u, sem_v):
    wid = lax.axis_index("s") * NUM_CORES + lax.axis_index("c")
    base_w = wid * E_PER_W
    lane = lax.iota(jnp.int32, LANES)

    def chunk_body(i, carry):
        base = base_w + i * CHUNK
        pltpu.sync_copy(src_hbm.at[pl.ds(base, CHUNK)], idx_u)
        pltpu.sync_copy(dst_hbm.at[pl.ds(base, CHUNK)], idx_v)
        cu = pltpu.async_copy(h_hbm.at[idx_u], rows_u, sem_u)
        cv = pltpu.async_copy(h_hbm.at[idx_v], rows_v, sem_v)
        cu.wait()
        cv.wait()

        def group_body(g, carry2):
            tot = jnp.zeros((LANES,), jnp.float32)
            for e in range(LANES):
                row = g * LANES + e
                acc = None
                for k in range(D_FEAT // LANES):
                    uu = rows_u[row, pl.ds(k * LANES, LANES)]
                    vv = rows_v[row, pl.ds(k * LANES, LANES)]
                    p = uu * vv
                    acc = p if acc is None else acc + p
                tot = jnp.where(lane == e, jnp.sum(acc), tot)
            score = 1.0 / (1.0 + jnp.exp(-tot))
            out_v[pl.ds(g * LANES, LANES)] = score
            return carry2

        lax.fori_loop(0, GROUPS, group_body, 0)
        pltpu.sync_copy(out_v, out_hbm.at[pl.ds(base, CHUNK)])
        return carry

    lax.fori_loop(0, N_CHUNKS, chunk_body, 0)


def kernel(h, edge_index):
    ei = edge_index.astype(jnp.int32)
    return _edge_dot_kernel(h, ei[0], ei[1])
